# Initial kernel scaffold; baseline (speedup 1.0000x reference)
import jax
import jax.numpy as jnp
from jax.experimental import pallas as pl


def kernel(xyz, feats, inter_idx, inter_w, W, gamma, beta):
    raise NotImplementedError("write your pallas kernel here")


# speedup vs baseline: 1165.3478x; 1165.3478x over previous
import functools

import jax
import jax.numpy as jnp
from jax import lax
from jax.experimental import pallas as pl
from jax.experimental.pallas import tpu as pltpu
from jax.experimental.pallas import tpu_sc as plsc

B = 4
N = 10000
M = 2500
MP = 2560
NN = 16
K = 3
A = 4
C = 64
O = 64
D = A * C
TOT = B * MP
KD = K * D

NW = 32
PER_W = TOT // NW
NB = 4
NCH = PER_W // NB
ROWS = NB * NN
WCH = NB * K * NN


def _sc_wsum(table, idx_flat, w_flat):
    mesh = plsc.VectorSubcoreMesh(core_axis_name="c", subcore_axis_name="s")
    info = plsc.get_sparse_core_info()
    nc = info.num_cores

    @functools.partial(
        pl.kernel,
        mesh=mesh,
        out_type=jax.ShapeDtypeStruct((TOT, KD), jnp.float32),
        scratch_types=[
            pltpu.VMEM((ROWS,), jnp.int32),
            pltpu.VMEM((ROWS,), jnp.int32),
            pltpu.VMEM((NB, K * NN, 16), jnp.float32),
            pltpu.VMEM((NB, K * NN, 16), jnp.float32),
            pltpu.VMEM((ROWS, D), jnp.float32),
            pltpu.VMEM((ROWS, D), jnp.float32),
            pltpu.VMEM((NB, KD), jnp.float32),
            pltpu.SemaphoreType.DMA,
            pltpu.SemaphoreType.DMA,
        ],
    )
    def k(table_hbm, idx_hbm, w_hbm, out_hbm,
          idx_v0, idx_v1, w_v0, w_v1, rows_v0, rows_v1, out_v, sem0, sem1):
        wid = lax.axis_index("s") * nc + lax.axis_index("c")
        base = wid * PER_W
        idx_bufs = (idx_v0, idx_v1)
        w_bufs = (w_v0, w_v1)
        row_bufs = (rows_v0, rows_v1)
        sems = (sem0, sem1)

        def stage(c, b):
            bm0 = base + c * NB
            pltpu.sync_copy(idx_hbm.at[pl.ds(bm0 * NN, ROWS)], idx_bufs[b])
            pltpu.sync_copy(w_hbm.at[pl.ds(bm0, NB)], w_bufs[b])
            pltpu.make_async_copy(
                table_hbm.at[idx_bufs[b]], row_bufs[b], sems[b]).start()

        stage(0, 0)
        stage(1, 1)

        def super_body(s, carry):
            for b in range(2):
                c = s * 2 + b
                rows_v = row_bufs[b]
                w_v = w_bufs[b]
                pltpu.make_async_copy(
                    table_hbm.at[idx_bufs[b]], rows_v, sems[b]).wait()

                def j_body(j, carry2):
                    for dg in range(2):
                        accs = [[jnp.zeros((16,), jnp.float32)
                                 for _ in range(8)] for _ in range(K)]
                        for n in range(16):
                            r = [rows_v[j * NN + n,
                                        pl.ds(dg * 128 + d * 16, 16)]
                                 for d in range(8)]
                            for kk in range(K):
                                wspl = w_v[j, kk * NN + n, :]
                                for d in range(8):
                                    accs[kk][d] = accs[kk][d] + wspl * r[d]
                        for kk in range(K):
                            for d in range(8):
                                out_v[j, pl.ds(kk * D + dg * 128 + d * 16,
                                               16)] = accs[kk][d]
                    return carry2

                lax.fori_loop(0, NB, j_body, 0)
                bm0 = base + c * NB
                pltpu.sync_copy(out_v, out_hbm.at[pl.ds(bm0, NB)])

                @pl.when(c + 2 < NCH)
                def _prefetch():
                    stage(c + 2, b)
            return carry

        lax.fori_loop(0, NCH // 2, super_body, 0)

    return k(table, idx_flat, w_flat)


_RB = 512
_GRID = TOT // _RB


def _mm_body(x_ref, w2_ref, y_ref, sum_ref, sq_ref):
    @pl.when(pl.program_id(0) == 0)
    def _init():
        sum_ref[...] = jnp.zeros_like(sum_ref)
        sq_ref[...] = jnp.zeros_like(sq_ref)

    y = jnp.dot(x_ref[...], w2_ref[...], preferred_element_type=jnp.float32)
    y_ref[...] = y
    sum_ref[...] += jnp.sum(y, axis=0, keepdims=True)
    sq_ref[...] += jnp.sum(y * y, axis=0, keepdims=True)


def _tc_matmul_stats(x, w2):
    return pl.pallas_call(
        _mm_body,
        grid=(_GRID,),
        in_specs=[
            pl.BlockSpec((_RB, KD), lambda i: (i, 0)),
            pl.BlockSpec((KD, D), lambda i: (0, 0)),
        ],
        out_specs=[
            pl.BlockSpec((_RB, D), lambda i: (i, 0)),
            pl.BlockSpec((1, D), lambda i: (0, 0)),
            pl.BlockSpec((1, D), lambda i: (0, 0)),
        ],
        out_shape=[
            jax.ShapeDtypeStruct((TOT, D), jnp.float32),
            jax.ShapeDtypeStruct((1, D), jnp.float32),
            jax.ShapeDtypeStruct((1, D), jnp.float32),
        ],
    )(x, w2)


def _bn_body(y_ref, sc_ref, sh_ref, o_ref):
    o_ref[...] = jnp.maximum(y_ref[...] * sc_ref[...] + sh_ref[...], 0.0)


def _tc_bn_relu(y, scale_col, shift_col):
    return pl.pallas_call(
        _bn_body,
        grid=(_GRID,),
        in_specs=[
            pl.BlockSpec((_RB, D), lambda i: (i, 0)),
            pl.BlockSpec((1, D), lambda i: (0, 0)),
            pl.BlockSpec((1, D), lambda i: (0, 0)),
        ],
        out_specs=pl.BlockSpec((_RB, D), lambda i: (i, 0)),
        out_shape=jax.ShapeDtypeStruct((TOT, D), jnp.float32),
    )(y, scale_col, shift_col)


def kernel(xyz, feats, inter_idx, inter_w, W, gamma, beta):
    table = feats.transpose(0, 3, 2, 1).reshape(B * N, D)
    idxp = jnp.pad(inter_idx.astype(jnp.int32), ((0, 0), (0, MP - M), (0, 0)))
    idxp = idxp + (jnp.arange(B, dtype=jnp.int32) * N)[:, None, None]
    idx_flat = idxp.reshape(TOT * NN)
    w_flat = jnp.broadcast_to(
        jnp.pad(inter_w, ((0, 0), (0, MP - M), (0, 0), (0, 0))
                ).reshape(TOT, K * NN)[:, :, None],
        (TOT, K * NN, 16))
    w2 = jnp.einsum('ock,ab->kacbo', W,
                    jnp.eye(A, dtype=jnp.float32)).reshape(KD, D)

    wsum = _sc_wsum(table, idx_flat, w_flat)

    y, colsum, colsq = _tc_matmul_stats(wsum, w2)

    cnt = jnp.float32(B * M * A)
    mean = colsum.reshape(A, O).sum(0) / cnt
    var = colsq.reshape(A, O).sum(0) / cnt - mean * mean
    scale_o = gamma * lax.rsqrt(var + 1e-5)
    shift_o = beta - mean * scale_o
    scale_col = jnp.tile(scale_o, A).reshape(1, D)
    shift_col = jnp.tile(shift_o, A).reshape(1, D)

    yn = _tc_bn_relu(y, scale_col, shift_col)

    feat = yn.reshape(B, MP, A, O)[:, :M].transpose(0, 3, 2, 1)
    new_xyz = xyz[:, :, ::A]
    return (inter_idx, inter_w, new_xyz, feat)


# speedup vs baseline: 1750.1872x; 1.5019x over previous
import functools

import jax
import jax.numpy as jnp
from jax import lax
from jax.experimental import pallas as pl
from jax.experimental.pallas import tpu as pltpu
from jax.experimental.pallas import tpu_sc as plsc

B = 4
N = 10000
M = 2500
MP = 2560
NN = 16
K = 3
A = 4
C = 64
O = 64
D = A * C
TOT = B * MP
KD = K * D

NW = 32
PER_W = TOT // NW
NB = 4
NCH = PER_W // NB
ROWS = NB * NN
WCH = NB * K * NN


def _sc_wsum(table, idx_flat, w_flat):
    mesh = plsc.VectorSubcoreMesh(core_axis_name="c", subcore_axis_name="s")
    info = plsc.get_sparse_core_info()
    nc = info.num_cores

    @functools.partial(
        pl.kernel,
        mesh=mesh,
        out_type=jax.ShapeDtypeStruct((TOT, KD), jnp.float32),
        scratch_types=[
            pltpu.VMEM((ROWS,), jnp.int32),
            pltpu.VMEM((ROWS,), jnp.int32),
            pltpu.VMEM((NB, K, 16), jnp.float32),
            pltpu.VMEM((NB, K, 16), jnp.float32),
            pltpu.VMEM((ROWS, D), jnp.float32),
            pltpu.VMEM((ROWS, D), jnp.float32),
            pltpu.VMEM((NB, KD), jnp.float32),
            pltpu.SemaphoreType.DMA,
            pltpu.SemaphoreType.DMA,
        ],
    )
    def k(table_hbm, idx_hbm, w_hbm, out_hbm,
          idx_v0, idx_v1, w_v0, w_v1, rows_v0, rows_v1, out_v, sem0, sem1):
        wid = lax.axis_index("s") * nc + lax.axis_index("c")
        base = wid * PER_W
        idx_bufs = (idx_v0, idx_v1)
        w_bufs = (w_v0, w_v1)
        row_bufs = (rows_v0, rows_v1)
        sems = (sem0, sem1)

        def stage(c, b):
            bm0 = base + c * NB
            pltpu.sync_copy(idx_hbm.at[pl.ds(bm0 * NN, ROWS)], idx_bufs[b])
            pltpu.sync_copy(w_hbm.at[pl.ds(bm0, NB)], w_bufs[b])
            pltpu.make_async_copy(
                table_hbm.at[idx_bufs[b]], row_bufs[b], sems[b]).start()

        stage(0, 0)
        stage(1, 1)

        def super_body(s, carry):
            for b in range(2):
                c = s * 2 + b
                rows_v = row_bufs[b]
                w_v = w_bufs[b]
                pltpu.make_async_copy(
                    table_hbm.at[idx_bufs[b]], rows_v, sems[b]).wait()

                def j_body(j, carry2):
                    wv = [w_v[j, kk, :] for kk in range(K)]
                    for dg in range(2):
                        accs = [[jnp.zeros((16,), jnp.float32)
                                 for _ in range(8)] for _ in range(K)]
                        for n in range(16):
                            r = [rows_v[j * NN + n,
                                        pl.ds(dg * 128 + d * 16, 16)]
                                 for d in range(8)]
                            for kk in range(K):
                                wspl = wv[kk].at[
                                    jnp.full((16,), n, jnp.int32)
                                ].get(mode="promise_in_bounds")
                                for d in range(8):
                                    accs[kk][d] = accs[kk][d] + wspl * r[d]
                        for kk in range(K):
                            for d in range(8):
                                out_v[j, pl.ds(kk * D + dg * 128 + d * 16,
                                               16)] = accs[kk][d]
                    return carry2

                lax.fori_loop(0, NB, j_body, 0)
                bm0 = base + c * NB
                pltpu.sync_copy(out_v, out_hbm.at[pl.ds(bm0, NB)])

                @pl.when(c + 2 < NCH)
                def _prefetch():
                    stage(c + 2, b)
            return carry

        lax.fori_loop(0, NCH // 2, super_body, 0)

    return k(table, idx_flat, w_flat)


def _tr_body(f_ref, t_ref):
    t_ref[...] = jnp.transpose(f_ref[0], (1, 0))


def _tc_table(feats):
    return pl.pallas_call(
        _tr_body,
        grid=(B, 2),
        in_specs=[pl.BlockSpec((1, D // 2, N), lambda b, j: (b, j, 0))],
        out_specs=pl.BlockSpec((N, D // 2), lambda b, j: (b, j)),
        out_shape=jax.ShapeDtypeStruct((B * N, D), jnp.float32),
    )(feats.reshape(B, D, N))


_RB = 512
_GRID = TOT // _RB


def _mm_body(x_ref, w2_ref, y_ref, sum_ref, sq_ref):
    @pl.when(pl.program_id(0) == 0)
    def _init():
        sum_ref[...] = jnp.zeros_like(sum_ref)
        sq_ref[...] = jnp.zeros_like(sq_ref)

    y = jnp.dot(x_ref[...], w2_ref[...], preferred_element_type=jnp.float32)
    y_ref[...] = y
    sum_ref[...] += jnp.sum(y, axis=0, keepdims=True)
    sq_ref[...] += jnp.sum(y * y, axis=0, keepdims=True)


def _tc_matmul_stats(x, w2):
    return pl.pallas_call(
        _mm_body,
        grid=(_GRID,),
        in_specs=[
            pl.BlockSpec((_RB, KD), lambda i: (i, 0)),
            pl.BlockSpec((KD, D), lambda i: (0, 0)),
        ],
        out_specs=[
            pl.BlockSpec((_RB, D), lambda i: (i, 0)),
            pl.BlockSpec((1, D), lambda i: (0, 0)),
            pl.BlockSpec((1, D), lambda i: (0, 0)),
        ],
        out_shape=[
            jax.ShapeDtypeStruct((TOT, D), jnp.float32),
            jax.ShapeDtypeStruct((1, D), jnp.float32),
            jax.ShapeDtypeStruct((1, D), jnp.float32),
        ],
    )(x, w2)


def _bn_body(y_ref, sc_ref, sh_ref, o_ref):
    z = jnp.maximum(y_ref[...] * sc_ref[...] + sh_ref[...], 0.0)
    t = jnp.transpose(z, (1, 0))
    o_ref[0] = t.reshape(O, A, MP)[:, :, :M]


def _tc_bn_relu(y, scale_col, shift_col):
    return pl.pallas_call(
        _bn_body,
        grid=(B,),
        in_specs=[
            pl.BlockSpec((MP, D), lambda b: (b, 0)),
            pl.BlockSpec((1, D), lambda b: (0, 0)),
            pl.BlockSpec((1, D), lambda b: (0, 0)),
        ],
        out_specs=pl.BlockSpec((1, O, A, M), lambda b: (b, 0, 0, 0)),
        out_shape=jax.ShapeDtypeStruct((B, O, A, M), jnp.float32),
    )(y, scale_col, shift_col)


def kernel(xyz, feats, inter_idx, inter_w, W, gamma, beta):
    table = _tc_table(feats)
    idxp = jnp.pad(inter_idx.astype(jnp.int32), ((0, 0), (0, MP - M), (0, 0)))
    idxp = idxp + (jnp.arange(B, dtype=jnp.int32) * N)[:, None, None]
    idx_flat = idxp.reshape(TOT * NN)
    w_flat = jnp.pad(inter_w, ((0, 0), (0, MP - M), (0, 0), (0, 0))
                     ).reshape(TOT, K, NN)
    w2 = jnp.einsum('ock,ab->kcaob', W,
                    jnp.eye(A, dtype=jnp.float32)).reshape(KD, D)

    wsum = _sc_wsum(table, idx_flat, w_flat)

    y, colsum, colsq = _tc_matmul_stats(wsum, w2)

    cnt = jnp.float32(B * M * A)
    mean = colsum.reshape(O, A).sum(1) / cnt
    var = colsq.reshape(O, A).sum(1) / cnt - mean * mean
    scale_o = gamma * lax.rsqrt(var + 1e-5)
    shift_o = beta - mean * scale_o
    scale_col = jnp.repeat(scale_o, A).reshape(1, D)
    shift_col = jnp.repeat(shift_o, A).reshape(1, D)

    feat = _tc_bn_relu(y, scale_col, shift_col)
    new_xyz = xyz[:, :, ::4]
    return (inter_idx, inter_w, new_xyz, feat)
